# Initial kernel scaffold; baseline (speedup 1.0000x reference)
import jax
import jax.numpy as jnp
from jax.experimental import pallas as pl


def kernel(features, edge_index_r0, edge_weight_r0, edge_index_r1, edge_weight_r1, self_proj, kernel_r0, kernel_r1, gamma, beta):
    raise NotImplementedError("write your pallas kernel here")


# speedup vs baseline: 2.9071x; 2.9071x over previous
import functools

import jax
import jax.numpy as jnp
from jax import lax
from jax.experimental import pallas as pl
from jax.experimental.pallas import tpu as pltpu
from jax.experimental.pallas import tpu_sc as plsc

N = 10000
F = 128
D = 128
E = 320000

CH = 128
NTILES = 32
C = 80
R = NTILES * C
NPAD = 10240
RPT = NPAD // 16

BN = 400
GRID = N // BN

_GDN = lax.GatherDimensionNumbers(
    offset_dims=(), collapsed_slice_dims=(0,), start_index_map=(0,))


def _matmul3(x, w_all):
    def body(x_ref, w_ref, h_ref, t0_ref, t1_ref):
        acc = jnp.dot(x_ref[...], w_ref[...], preferred_element_type=jnp.float32)
        h_ref[...] = acc[:, 0:D]
        t0_ref[...] = acc[:, D:2 * D]
        t1_ref[...] = acc[:, 2 * D:3 * D]

    return pl.pallas_call(
        body,
        grid=(GRID,),
        in_specs=[
            pl.BlockSpec((BN, F), lambda i: (i, 0)),
            pl.BlockSpec((F, 3 * D), lambda i: (0, 0)),
        ],
        out_specs=[
            pl.BlockSpec((BN, D), lambda i: (i, 0)),
            pl.BlockSpec((BN, D), lambda i: (i, 0)),
            pl.BlockSpec((BN, D), lambda i: (i, 0)),
        ],
        out_shape=[
            jax.ShapeDtypeStruct((N, D), jnp.float32),
            jax.ShapeDtypeStruct((N, D), jnp.float32),
            jax.ShapeDtypeStruct((N, D), jnp.float32),
        ],
    )(x, w_all)


@functools.partial(
    pl.kernel,
    out_type=[jax.ShapeDtypeStruct((NPAD, D), jnp.float32),
              jax.ShapeDtypeStruct((NPAD, D), jnp.float32)],
    mesh=plsc.VectorSubcoreMesh(core_axis_name="c", subcore_axis_name="s"),
    scratch_types=[
        pltpu.VMEM((C, CH), jnp.int32),
        pltpu.VMEM((C, CH), jnp.int32),
        pltpu.VMEM((C * CH,), jnp.float32),
        pltpu.VMEM((CH, D), jnp.float32),
        pltpu.VMEM_SHARED((NPAD, D), jnp.float32),
        pltpu.SemaphoreType.DMA,
    ],
)
def _sc_edge(t0_hbm, t1_hbm, s0, d0, w0, s1, d1, w1, out0_hbm, out1_hbm,
             src_v, dst_v, ew_v, rows_v, agg, gsem):
    cid = lax.axis_index("c")
    sid = lax.axis_index("s")
    tile = sid * 2 + cid

    def zrow(i, _):
        for g in range(8):
            rows_v[i, pl.ds(g * 16, 16)] = jnp.zeros((16,), jnp.float32)
        return 0
    lax.fori_loop(0, CH, zrow, 0)
    for z in range(RPT // CH):
        pltpu.sync_copy(rows_v, agg.at[pl.ds(sid * RPT + z * CH, CH)])
    plsc.subcore_barrier()

    for (t_hbm, s_h, d_h, w_h) in ((t0_hbm, s0, d0, w0), (t1_hbm, s1, d1, w1)):
        pltpu.sync_copy(s_h.at[pl.ds(tile * C, C)], src_v)
        pltpu.sync_copy(d_h.at[pl.ds(tile * C, C)], dst_v)
        pltpu.sync_copy(w_h.at[pl.ds(tile * C * CH, C * CH)], ew_v)

        def chunk_body(j, _):
            pltpu.async_copy(t_hbm.at[src_v.at[j]], rows_v, gsem).wait()

            def escale(g16, __):
                ew16 = ew_v[pl.ds(j * CH + g16 * 16, 16)]
                for l in range(16):
                    lidx = lax.broadcast_in_dim(
                        jnp.int32(l), (16, 1), ())
                    w16 = lax.gather(
                        ew16, lidx,
                        dimension_numbers=_GDN, slice_sizes=(1,),
                        mode=lax.GatherScatterMode.PROMISE_IN_BOUNDS)
                    e = g16 * 16 + l
                    for g in range(8):
                        sl = pl.ds(g * 16, 16)
                        rows_v[e, sl] = rows_v[e, sl] * w16
                return 0
            lax.fori_loop(0, CH // 16, escale, 0)

            pltpu.sync_copy(rows_v, agg.at[dst_v.at[j]], add=True)
            return 0
        lax.fori_loop(0, C, chunk_body, 0)

    plsc.subcore_barrier()
    @pl.when(cid == 0)
    def _():
        pltpu.sync_copy(agg.at[pl.ds(sid * RPT, RPT)],
                        out0_hbm.at[pl.ds(sid * RPT, RPT)])
    @pl.when(cid == 1)
    def _():
        pltpu.sync_copy(agg.at[pl.ds(sid * RPT, RPT)],
                        out1_hbm.at[pl.ds(sid * RPT, RPT)])


def _finish(h_self, p0, p1, gamma2, beta2):
    def body(h_ref, p0_ref, p1_ref, g_ref, b_ref, o_ref):
        s = h_ref[...] + p0_ref[...] + p1_ref[...]
        mean = jnp.mean(s, axis=1, keepdims=True)
        c = s - mean
        var = jnp.mean(c * c, axis=1, keepdims=True)
        normed = c * lax.rsqrt(var + 1e-3)
        o_ref[...] = jnp.maximum(normed * g_ref[...] + b_ref[...], 0.0)

    return pl.pallas_call(
        body,
        grid=(GRID,),
        in_specs=[
            pl.BlockSpec((BN, D), lambda i: (i, 0)),
            pl.BlockSpec((BN, D), lambda i: (i, 0)),
            pl.BlockSpec((BN, D), lambda i: (i, 0)),
            pl.BlockSpec((1, D), lambda i: (0, 0)),
            pl.BlockSpec((1, D), lambda i: (0, 0)),
        ],
        out_specs=pl.BlockSpec((BN, D), lambda i: (i, 0)),
        out_shape=jax.ShapeDtypeStruct((N, D), jnp.float32),
    )(h_self, p0, p1, gamma2, beta2)


def _prep_edges(ei, ew):
    pad = R * CH - E
    src = jnp.pad(ei[0], (0, pad)).reshape(R, CH)
    dst = jnp.pad(ei[1], (0, pad)).reshape(R, CH)
    eww = jnp.pad(ew, (0, pad))
    return src, dst, eww


def kernel(features, edge_index_r0, edge_weight_r0, edge_index_r1,
           edge_weight_r1, self_proj, kernel_r0, kernel_r1, gamma, beta):
    x = features.reshape(N, F)
    w_all = jnp.concatenate([self_proj, kernel_r0, kernel_r1], axis=1)
    h_self, t0, t1 = _matmul3(x, w_all)

    s0, d0, w0 = _prep_edges(edge_index_r0, edge_weight_r0)
    s1, d1, w1 = _prep_edges(edge_index_r1, edge_weight_r1)

    p0, p1 = _sc_edge(t0, t1, s0, d0, w0, s1, d1, w1)

    out = _finish(h_self, p0, p1, gamma.reshape(1, D), beta.reshape(1, D))
    return out.reshape(1, N, D)


# speedup vs baseline: 3.3448x; 1.1506x over previous
import functools

import jax
import jax.numpy as jnp
from jax import lax
from jax.experimental import pallas as pl
from jax.experimental.pallas import tpu as pltpu
from jax.experimental.pallas import tpu_sc as plsc

N = 10000
F = 128
D = 128
E = 320000

CH = 128
NTILES = 32
C = 80
GC = 16
R = NTILES * C
NPAD = 10240
RPT = NPAD // 16

BN = 400
GRID = N // BN

_GDN = lax.GatherDimensionNumbers(
    offset_dims=(), collapsed_slice_dims=(0,), start_index_map=(0,))


def _matmul3(x, w_all):
    def body(x_ref, w_ref, h_ref, t0_ref, t1_ref):
        acc = jnp.dot(x_ref[...], w_ref[...], preferred_element_type=jnp.float32)
        h_ref[...] = acc[:, 0:D]
        t0_ref[...] = acc[:, D:2 * D]
        t1_ref[...] = acc[:, 2 * D:3 * D]

    return pl.pallas_call(
        body,
        grid=(GRID,),
        in_specs=[
            pl.BlockSpec((BN, F), lambda i: (i, 0)),
            pl.BlockSpec((F, 3 * D), lambda i: (0, 0)),
        ],
        out_specs=[
            pl.BlockSpec((BN, D), lambda i: (i, 0)),
            pl.BlockSpec((BN, D), lambda i: (i, 0)),
            pl.BlockSpec((BN, D), lambda i: (i, 0)),
        ],
        out_shape=[
            jax.ShapeDtypeStruct((N, D), jnp.float32),
            jax.ShapeDtypeStruct((N, D), jnp.float32),
            jax.ShapeDtypeStruct((N, D), jnp.float32),
        ],
    )(x, w_all)


@functools.partial(
    pl.kernel,
    out_type=[jax.ShapeDtypeStruct((NPAD, D), jnp.float32),
              jax.ShapeDtypeStruct((NPAD, D), jnp.float32)],
    mesh=plsc.VectorSubcoreMesh(core_axis_name="c", subcore_axis_name="s"),
    scratch_types=[
        pltpu.VMEM((GC, CH), jnp.int32),
        pltpu.VMEM((GC, CH), jnp.int32),
        pltpu.VMEM((GC * CH,), jnp.float32),
        pltpu.VMEM((CH, D), jnp.float32),
        pltpu.VMEM((CH, D), jnp.float32),
        pltpu.VMEM_SHARED((NPAD, D), jnp.float32),
        pltpu.SemaphoreType.DMA,
        pltpu.SemaphoreType.DMA,
        pltpu.SemaphoreType.DMA,
        pltpu.SemaphoreType.DMA,
    ],
)
def _sc_edge(t0_hbm, t1_hbm, s0, d0, w0, s1, d1, w1, out0_hbm, out1_hbm,
             src_v, dst_v, ew_v, r0, r1, agg, g0, g1, a0, a1):
    cid = lax.axis_index("c")
    sid = lax.axis_index("s")
    tile = sid * 2 + cid

    rows = (r0, r1)
    gsem = (g0, g1)
    asem = (a0, a1)
    NB = 2

    def zrow(i, _):
        for g in range(8):
            r0[i, pl.ds(g * 16, 16)] = jnp.zeros((16,), jnp.float32)
        return 0
    lax.fori_loop(0, CH, zrow, 0)
    for z in range(RPT // CH):
        pltpu.sync_copy(r0, agg.at[pl.ds(sid * RPT + z * CH, CH)])
    plsc.subcore_barrier()

    def scale(buf, j):
        def escale(g16, __):
            ew16 = ew_v[pl.ds(j * CH + g16 * 16, 16)]
            for l in range(16):
                lidx = lax.broadcast_in_dim(jnp.int32(l), (16, 1), ())
                w16 = lax.gather(
                    ew16, lidx,
                    dimension_numbers=_GDN, slice_sizes=(1,),
                    mode=lax.GatherScatterMode.PROMISE_IN_BOUNDS)
                e = g16 * 16 + l
                for g in range(8):
                    sl = pl.ds(g * 16, 16)
                    buf[e, sl] = buf[e, sl] * w16
            return 0
        lax.fori_loop(0, CH // 16, escale, 0)

    for (t_hbm, s_h, d_h, w_h) in ((t0_hbm, s0, d0, w0), (t1_hbm, s1, d1, w1)):

        def start_gather(x, j):
            pltpu.make_async_copy(
                t_hbm.at[src_v.at[j]], rows[x], gsem[x]).start()

        def wait_gather(x, j):
            pltpu.make_async_copy(
                t_hbm.at[src_v.at[j]], rows[x], gsem[x]).wait()

        def start_scatter(x, j):
            pltpu.async_copy(rows[x], agg.at[dst_v.at[j]], asem[x], add=True)

        def wait_scatter(x, j):
            pltpu.make_async_copy(
                rows[x], agg.at[dst_v.at[j]], asem[x]).wait()

        def group_body(grp, _):
            pltpu.sync_copy(s_h.at[pl.ds(tile * C + grp * GC, GC)], src_v)
            pltpu.sync_copy(d_h.at[pl.ds(tile * C + grp * GC, GC)], dst_v)
            pltpu.sync_copy(
                w_h.at[pl.ds((tile * C + grp * GC) * CH, GC * CH)], ew_v)

            for x in range(NB):
                start_gather(x, x)

            def body(j2, __):
                base = j2 * NB
                for x in range(NB):
                    wait_gather(x, base + x)
                    scale(rows[x], base + x)
                    start_scatter(x, base + x)
                for x in range(NB):
                    wait_scatter(x, base + x)
                    start_gather(x, base + NB + x)
                return 0
            lax.fori_loop(0, GC // NB - 1, body, 0)

            last = GC - NB
            for x in range(NB):
                wait_gather(x, last + x)
                scale(rows[x], last + x)
                start_scatter(x, last + x)
            for x in range(NB):
                wait_scatter(x, last + x)
            return 0
        lax.fori_loop(0, C // GC, group_body, 0)

    plsc.subcore_barrier()
    @pl.when(cid == 0)
    def _():
        pltpu.sync_copy(agg.at[pl.ds(sid * RPT, RPT)],
                        out0_hbm.at[pl.ds(sid * RPT, RPT)])
    @pl.when(cid == 1)
    def _():
        pltpu.sync_copy(agg.at[pl.ds(sid * RPT, RPT)],
                        out1_hbm.at[pl.ds(sid * RPT, RPT)])


def _finish(h_self, p0, p1, gamma2, beta2):
    def body(h_ref, p0_ref, p1_ref, g_ref, b_ref, o_ref):
        s = h_ref[...] + p0_ref[...] + p1_ref[...]
        mean = jnp.mean(s, axis=1, keepdims=True)
        c = s - mean
        var = jnp.mean(c * c, axis=1, keepdims=True)
        normed = c * lax.rsqrt(var + 1e-3)
        o_ref[...] = jnp.maximum(normed * g_ref[...] + b_ref[...], 0.0)

    return pl.pallas_call(
        body,
        grid=(GRID,),
        in_specs=[
            pl.BlockSpec((BN, D), lambda i: (i, 0)),
            pl.BlockSpec((BN, D), lambda i: (i, 0)),
            pl.BlockSpec((BN, D), lambda i: (i, 0)),
            pl.BlockSpec((1, D), lambda i: (0, 0)),
            pl.BlockSpec((1, D), lambda i: (0, 0)),
        ],
        out_specs=pl.BlockSpec((BN, D), lambda i: (i, 0)),
        out_shape=jax.ShapeDtypeStruct((N, D), jnp.float32),
    )(h_self, p0, p1, gamma2, beta2)


def _prep_edges(ei, ew):
    pad = R * CH - E
    src = jnp.pad(ei[0], (0, pad)).reshape(R, CH)
    dst = jnp.pad(ei[1], (0, pad)).reshape(R, CH)
    eww = jnp.pad(ew, (0, pad))
    return src, dst, eww


def kernel(features, edge_index_r0, edge_weight_r0, edge_index_r1,
           edge_weight_r1, self_proj, kernel_r0, kernel_r1, gamma, beta):
    x = features.reshape(N, F)
    w_all = jnp.concatenate([self_proj, kernel_r0, kernel_r1], axis=1)
    h_self, t0, t1 = _matmul3(x, w_all)

    s0, d0, w0 = _prep_edges(edge_index_r0, edge_weight_r0)
    s1, d1, w1 = _prep_edges(edge_index_r1, edge_weight_r1)

    p0, p1 = _sc_edge(t0, t1, s0, d0, w0, s1, d1, w1)

    out = _finish(h_self, p0, p1, gamma.reshape(1, D), beta.reshape(1, D))
    return out.reshape(1, N, D)


# speedup vs baseline: 3.7930x; 1.1340x over previous
import functools

import jax
import jax.numpy as jnp
from jax import lax
from jax.experimental import pallas as pl
from jax.experimental.pallas import tpu as pltpu
from jax.experimental.pallas import tpu_sc as plsc

N = 10000
F = 128
D = 128
E = 320000

CH = 128
NTILES = 32
C = 80
CF = 120
CS = 40
GC = 8
R = NTILES * C
NPAD = 10240
RPT = NPAD // 16

BN = 400
GRID = N // BN

_GDN = lax.GatherDimensionNumbers(
    offset_dims=(), collapsed_slice_dims=(0,), start_index_map=(0,))


def _matmul3(x, w_all):
    def body(x_ref, w_ref, h_ref, t0_ref, t1_ref):
        acc = jnp.dot(x_ref[...], w_ref[...], preferred_element_type=jnp.float32)
        h_ref[...] = acc[:, 0:D]
        t0_ref[...] = acc[:, D:2 * D]
        t1_ref[...] = acc[:, 2 * D:3 * D]

    return pl.pallas_call(
        body,
        grid=(GRID,),
        in_specs=[
            pl.BlockSpec((BN, F), lambda i: (i, 0)),
            pl.BlockSpec((F, 3 * D), lambda i: (0, 0)),
        ],
        out_specs=[
            pl.BlockSpec((BN, D), lambda i: (i, 0)),
            pl.BlockSpec((BN, D), lambda i: (i, 0)),
            pl.BlockSpec((BN, D), lambda i: (i, 0)),
        ],
        out_shape=[
            jax.ShapeDtypeStruct((N, D), jnp.float32),
            jax.ShapeDtypeStruct((N, D), jnp.float32),
            jax.ShapeDtypeStruct((N, D), jnp.float32),
        ],
    )(x, w_all)


@functools.partial(
    pl.kernel,
    out_type=[jax.ShapeDtypeStruct((NPAD, D), jnp.float32),
              jax.ShapeDtypeStruct((NPAD, D), jnp.float32)],
    mesh=plsc.VectorSubcoreMesh(core_axis_name="c", subcore_axis_name="s"),
    scratch_types=[
        pltpu.VMEM((GC, CH), jnp.int32),
        pltpu.VMEM((GC, CH), jnp.int32),
        pltpu.VMEM((GC * CH,), jnp.float32),
        pltpu.VMEM((CH, D), jnp.float32),
        pltpu.VMEM((CH, D), jnp.float32),
        pltpu.VMEM_SHARED((NPAD, D), jnp.float32),
        pltpu.SemaphoreType.DMA,
        pltpu.SemaphoreType.DMA,
        pltpu.SemaphoreType.DMA,
        pltpu.SemaphoreType.DMA,
    ],
)
def _sc_edge(t0_hbm, t1_hbm, s0, d0, w0, s1, d1, w1, out0_hbm, out1_hbm,
             src_v, dst_v, ew_v, r0, r1, agg, g0, g1, a0, a1):
    cid = lax.axis_index("c")
    sid = lax.axis_index("s")
    n_groups = jnp.where(cid == 0, CF // GC, CS // GC)
    row_base = jnp.where(cid == 0, sid * CF, 16 * CF + sid * CS)

    rows = (r0, r1)
    gsem = (g0, g1)
    asem = (a0, a1)
    NB = 2

    def zrow(i, _):
        for g in range(8):
            r0[i, pl.ds(g * 16, 16)] = jnp.zeros((16,), jnp.float32)
        return 0
    lax.fori_loop(0, CH, zrow, 0)
    for z in range(RPT // CH):
        pltpu.sync_copy(r0, agg.at[pl.ds(sid * RPT + z * CH, CH)])
    plsc.subcore_barrier()

    def scale(buf, j):
        def escale(g16, __):
            ew16 = ew_v[pl.ds(j * CH + g16 * 16, 16)]
            for l in range(16):
                lidx = lax.broadcast_in_dim(jnp.int32(l), (16, 1), ())
                w16 = lax.gather(
                    ew16, lidx,
                    dimension_numbers=_GDN, slice_sizes=(1,),
                    mode=lax.GatherScatterMode.PROMISE_IN_BOUNDS)
                e = g16 * 16 + l
                for g in range(8):
                    sl = pl.ds(g * 16, 16)
                    buf[e, sl] = buf[e, sl] * w16
            return 0
        lax.fori_loop(0, CH // 16, escale, 0)

    for (t_hbm, s_h, d_h, w_h) in ((t0_hbm, s0, d0, w0), (t1_hbm, s1, d1, w1)):

        def start_gather(x, j):
            pltpu.make_async_copy(
                t_hbm.at[src_v.at[j]], rows[x], gsem[x]).start()

        def wait_gather(x, j):
            pltpu.make_async_copy(
                t_hbm.at[src_v.at[j]], rows[x], gsem[x]).wait()

        def start_scatter(x, j):
            pltpu.async_copy(rows[x], agg.at[dst_v.at[j]], asem[x], add=True)

        def wait_scatter(x, j):
            pltpu.make_async_copy(
                rows[x], agg.at[dst_v.at[j]], asem[x]).wait()

        def group_body(grp, _):
            g_row = row_base + grp * GC
            pltpu.sync_copy(s_h.at[pl.ds(g_row, GC)], src_v)
            pltpu.sync_copy(d_h.at[pl.ds(g_row, GC)], dst_v)
            pltpu.sync_copy(w_h.at[pl.ds(g_row * CH, GC * CH)], ew_v)

            for x in range(NB):
                start_gather(x, x)

            def body(j2, __):
                base = j2 * NB
                for x in range(NB):
                    wait_gather(x, base + x)
                    scale(rows[x], base + x)
                    start_scatter(x, base + x)
                for x in range(NB):
                    wait_scatter(x, base + x)
                    start_gather(x, base + NB + x)
                return 0
            lax.fori_loop(0, GC // NB - 1, body, 0)

            last = GC - NB
            for x in range(NB):
                wait_gather(x, last + x)
                scale(rows[x], last + x)
                start_scatter(x, last + x)
            for x in range(NB):
                wait_scatter(x, last + x)
            return 0
        lax.fori_loop(0, n_groups, group_body, 0)

    plsc.subcore_barrier()
    @pl.when(cid == 0)
    def _():
        pltpu.sync_copy(agg.at[pl.ds(sid * RPT, RPT)],
                        out0_hbm.at[pl.ds(sid * RPT, RPT)])
    @pl.when(cid == 1)
    def _():
        pltpu.sync_copy(agg.at[pl.ds(sid * RPT, RPT)],
                        out1_hbm.at[pl.ds(sid * RPT, RPT)])


def _finish(h_self, p0, p1, gamma2, beta2):
    def body(h_ref, p0_ref, p1_ref, g_ref, b_ref, o_ref):
        s = h_ref[...] + p0_ref[...] + p1_ref[...]
        mean = jnp.mean(s, axis=1, keepdims=True)
        c = s - mean
        var = jnp.mean(c * c, axis=1, keepdims=True)
        normed = c * lax.rsqrt(var + 1e-3)
        o_ref[...] = jnp.maximum(normed * g_ref[...] + b_ref[...], 0.0)

    return pl.pallas_call(
        body,
        grid=(GRID,),
        in_specs=[
            pl.BlockSpec((BN, D), lambda i: (i, 0)),
            pl.BlockSpec((BN, D), lambda i: (i, 0)),
            pl.BlockSpec((BN, D), lambda i: (i, 0)),
            pl.BlockSpec((1, D), lambda i: (0, 0)),
            pl.BlockSpec((1, D), lambda i: (0, 0)),
        ],
        out_specs=pl.BlockSpec((BN, D), lambda i: (i, 0)),
        out_shape=jax.ShapeDtypeStruct((N, D), jnp.float32),
    )(h_self, p0, p1, gamma2, beta2)


def _prep_edges(ei, ew):
    pad = R * CH - E
    src = jnp.pad(ei[0], (0, pad)).reshape(R, CH)
    dst = jnp.pad(ei[1], (0, pad)).reshape(R, CH)
    eww = jnp.pad(ew, (0, pad))
    return src, dst, eww


def kernel(features, edge_index_r0, edge_weight_r0, edge_index_r1,
           edge_weight_r1, self_proj, kernel_r0, kernel_r1, gamma, beta):
    x = features.reshape(N, F)
    w_all = jnp.concatenate([self_proj, kernel_r0, kernel_r1], axis=1)
    h_self, t0, t1 = _matmul3(x, w_all)

    s0, d0, w0 = _prep_edges(edge_index_r0, edge_weight_r0)
    s1, d1, w1 = _prep_edges(edge_index_r1, edge_weight_r1)

    p0, p1 = _sc_edge(t0, t1, s0, d0, w0, s1, d1, w1)

    out = _finish(h_self, p0, p1, gamma.reshape(1, D), beta.reshape(1, D))
    return out.reshape(1, N, D)
